# baseline (speedup 1.0000x reference)
import jax
import jax.numpy as jnp
from jax.experimental import pallas as pl
from jax.experimental.pallas import tpu as pltpu
from jax.sharding import Mesh, PartitionSpec as P


def _copy_body(x_ref, o_ref):
    o_ref[...] = x_ref[...]


def _per_dev(xs):
    hw_s, N, C = xs.shape
    t = hw_s // 7
    return pl.pallas_call(
        _copy_body,
        out_shape=jax.ShapeDtypeStruct((hw_s, N, C), xs.dtype),
        grid=(7,),
        in_specs=[pl.BlockSpec((t, N, C), lambda k: (k, 0, 0))],
        out_specs=pl.BlockSpec((t, N, C), lambda k: (k, 0, 0)),
        compiler_params=pltpu.CompilerParams(
            dimension_semantics=("parallel",),
            vmem_limit_bytes=60 << 20),
    )(xs)


def kernel(x, w1, b1, w2, b2):
    N, C, H, W = x.shape
    HW = H * W
    xt = jnp.transpose(x, (2, 3, 0, 1)).reshape(HW, N, C)
    devs = jax.devices()[:2]
    mesh = Mesh(devs, ("d",))
    out_t = jax.shard_map(
        _per_dev, mesh=mesh, in_specs=P("d", None, None),
        out_specs=P("d", None, None), check_vma=False)(xt)
    return jnp.transpose(out_t.reshape(H, W, N, C), (2, 3, 0, 1))


# speedup vs baseline: 11.6293x; 11.6293x over previous
import functools

import jax
import jax.numpy as jnp
from jax.experimental import pallas as pl
from jax.experimental.pallas import tpu as pltpu


def _se_body(x_hbm, w1t_ref, b1_ref, w2t_ref, b2_ref, o_hbm,
             buf, acc, in_sems, out_sems, *, inv_hw, in_offs, out_offs,
             n_sub, splits):
    num_in = len(in_offs) - 1
    num_out = len(out_offs) - 1

    def chunk_in(part_i, k):
        o, n = in_offs[k], in_offs[k + 1] - in_offs[k]
        sl = pl.ds(part_i * n_sub, n_sub)
        return pltpu.make_async_copy(
            x_hbm.at[pl.ds(o, n), sl], buf.at[pl.ds(o, n), sl],
            in_sems.at[part_i, k])

    def chunk_out(part_i, k):
        o, n = out_offs[k], out_offs[k + 1] - out_offs[k]
        sl = pl.ds(part_i * n_sub, n_sub)
        return pltpu.make_async_copy(
            buf.at[pl.ds(o, n), sl], o_hbm.at[pl.ds(o, n), sl],
            out_sems.at[part_i])

    for part_i in range(splits):
        for k in range(num_in):
            chunk_in(part_i, k).start()

    for part_i in range(splits):
        nsl = pl.ds(part_i * n_sub, n_sub)
        for k in range(num_in):
            chunk_in(part_i, k).wait()
            o, n = in_offs[k], in_offs[k + 1] - in_offs[k]
            part = jnp.sum(buf[pl.ds(o, n), nsl].astype(jnp.float32), axis=0)
            if k == 0:
                acc[...] = part
            else:
                acc[...] += part

        p = acc[...] * inv_hw
        h = jnp.maximum(
            jnp.dot(p, w1t_ref[...], preferred_element_type=jnp.float32)
            + b1_ref[...], 0.0)
        s = jax.nn.sigmoid(
            jnp.dot(h, w2t_ref[...], preferred_element_type=jnp.float32)
            + b2_ref[...])
        s = s[None].astype(buf.dtype)

        for k in range(num_out):
            sl = pl.ds(out_offs[k], out_offs[k + 1] - out_offs[k])
            buf[sl, nsl] = buf[sl, nsl] * s
            chunk_out(part_i, k).start()

    for part_i in range(splits):
        sl = pl.ds(part_i * n_sub, n_sub)
        pltpu.make_async_copy(
            buf.at[:, sl], o_hbm.at[:, sl], out_sems.at[part_i]).wait()


def _chunk_plan(hw: int, plane_bytes: int):
    base = 1
    for t in range(1, hw + 1):
        if hw % t == 0 and t * plane_bytes <= (8 << 20):
            base = t
    offs = list(range(0, hw + 1, base))
    quarter = max(1, base // 4)
    in_offs = list(offs)
    if base > 1:
        in_offs.insert(-1, hw - quarter)
    out_offs = list(offs)
    if base > 1:
        out_offs.insert(1, quarter)
    return in_offs, out_offs


def kernel(x, w1, b1, w2, b2):
    N, C, H, W = x.shape
    HW = H * W
    Cmid = w1.shape[0]
    dtype = x.dtype

    w1t = jnp.asarray(w1, jnp.float32).T.reshape(C, Cmid)
    b1r = jnp.asarray(b1, jnp.float32).reshape(1, Cmid)
    w2t = jnp.asarray(w2, jnp.float32).T.reshape(Cmid, C)
    b2r = jnp.asarray(b2, jnp.float32).reshape(1, C)

    xt = jnp.transpose(x, (2, 3, 0, 1)).reshape(HW, N, C)

    itemsize = jnp.dtype(dtype).itemsize
    splits = 4
    n_sub = N // splits
    plane_bytes = n_sub * C * itemsize
    in_offs, out_offs = _chunk_plan(HW, plane_bytes)

    body = functools.partial(_se_body, inv_hw=1.0 / float(HW),
                             in_offs=tuple(in_offs), out_offs=tuple(out_offs),
                             n_sub=n_sub, splits=splits)
    out_t = pl.pallas_call(
        body,
        out_shape=jax.ShapeDtypeStruct((HW, N, C), dtype),
        in_specs=[
            pl.BlockSpec(memory_space=pltpu.MemorySpace.HBM),
            pl.BlockSpec(memory_space=pltpu.MemorySpace.VMEM),
            pl.BlockSpec(memory_space=pltpu.MemorySpace.VMEM),
            pl.BlockSpec(memory_space=pltpu.MemorySpace.VMEM),
            pl.BlockSpec(memory_space=pltpu.MemorySpace.VMEM),
        ],
        out_specs=pl.BlockSpec(memory_space=pltpu.MemorySpace.HBM),
        scratch_shapes=[
            pltpu.VMEM((HW, N, C), dtype),
            pltpu.VMEM((n_sub, C), jnp.float32),
            pltpu.SemaphoreType.DMA((splits, len(in_offs) - 1)),
            pltpu.SemaphoreType.DMA((splits,)),
        ],
        compiler_params=pltpu.CompilerParams(
            vmem_limit_bytes=62 << 20),
    )(xt, w1t, b1r, w2t, b2r)

    return jnp.transpose(out_t.reshape(H, W, N, C), (2, 3, 0, 1))
